# Initial kernel scaffold; baseline (speedup 1.0000x reference)
import jax
import jax.numpy as jnp
from jax.experimental import pallas as pl


def kernel(features, labels, centers):
    raise NotImplementedError("write your pallas kernel here")


# speedup vs baseline: 3.4192x; 3.4192x over previous
import jax
import jax.numpy as jnp
from jax import lax
from jax.experimental import pallas as pl
from jax.experimental.pallas import tpu as pltpu
from jax.experimental.pallas import tpu_sc as plsc

_NUM_CLASSES = 8192
_DIM = 256
_ALPHA = 0.5
_BATCH = 16384

_NC = 2
_NS = 16
_COLS = _DIM // _NC
_HALF = _NUM_CLASSES // _NC
_ROWS_PER_TILE = _BATCH // _NS
_CHUNK = 64
_NCHUNK = _ROWS_PER_TILE // _CHUNK
_CNT_ROWS = _HALF + _NS
_FS_WPT = _NUM_CLASSES // _NS
_CNT_WPT = _HALF // _NS


def _sc_body(features, labels, zeros_b, ones_b,
             featsum_out, counts_out,
             idx_ref, idx2_ref, fbuf, ones_ref, featsum_sh, counts_sh):
    c = lax.axis_index("c")
    s = lax.axis_index("s")
    col0 = c * _COLS
    base = c * _HALF
    row0 = s * _ROWS_PER_TILE
    trash = _HALF + s

    pltpu.sync_copy(ones_b, ones_ref)
    pltpu.sync_copy(zeros_b, featsum_sh.at[pl.ds(s * _FS_WPT, _FS_WPT)])
    pltpu.sync_copy(zeros_b.at[pl.ds(0, _CNT_WPT)],
                    counts_sh.at[pl.ds(s * _CNT_WPT, _CNT_WPT)])

    @pl.when(s == _NS - 1)
    def _zero_trash():
        pltpu.sync_copy(zeros_b.at[pl.ds(0, _NS)],
                        counts_sh.at[pl.ds(_HALF, _NS)])

    plsc.subcore_barrier()

    for j in range(_NCHUNK):
        pltpu.sync_copy(labels.at[pl.ds(row0 + j * _CHUNK, _CHUNK)], idx_ref)
        pltpu.sync_copy(
            features.at[pl.ds(row0 + j * _CHUNK, _CHUNK), pl.ds(col0, _COLS)],
            fbuf)
        for k in range(_CHUNK // 16):
            lbl = idx_ref[pl.ds(k * 16, 16)]
            rel = lbl - base
            inr = jnp.logical_and(rel >= 0, rel < _HALF)
            idx2_ref[pl.ds(k * 16, 16)] = jnp.where(inr, rel, trash)
        pltpu.sync_copy(fbuf, featsum_sh.at[idx_ref], add=True)
        pltpu.sync_copy(ones_ref, counts_sh.at[idx2_ref], add=True)

    plsc.subcore_barrier()

    pltpu.sync_copy(featsum_sh.at[pl.ds(s * _FS_WPT, _FS_WPT)],
                    featsum_out.at[c, pl.ds(s * _FS_WPT, _FS_WPT)])
    pltpu.sync_copy(counts_sh.at[pl.ds(s * _CNT_WPT, _CNT_WPT)],
                    counts_out.at[c, pl.ds(s * _CNT_WPT, _CNT_WPT)])


_sc_segsum = pl.kernel(
    _sc_body,
    out_type=(
        jax.ShapeDtypeStruct((_NC, _NUM_CLASSES, _COLS), jnp.float32),
        jax.ShapeDtypeStruct((_NC, _HALF, 128), jnp.float32),
    ),
    mesh=plsc.VectorSubcoreMesh(core_axis_name="c", subcore_axis_name="s"),
    scratch_types=[
        pltpu.VMEM((_CHUNK,), jnp.int32),
        pltpu.VMEM((_CHUNK,), jnp.int32),
        pltpu.VMEM((_CHUNK, _COLS), jnp.float32),
        pltpu.VMEM((_CHUNK, 128), jnp.float32),
        pltpu.VMEM_SHARED((_NUM_CLASSES, _COLS), jnp.float32),
        pltpu.VMEM_SHARED((_CNT_ROWS, 128), jnp.float32),
    ],
)


_G = 16
_CB = _NUM_CLASSES // _G
_FB = _BATCH // _G


def _dense_body(counts_ref, f0_ref, f1_ref, centers_ref, features_ref,
                nc_ref, loss_ref, acc_ref):
    i = pl.program_id(0)
    cnt = counts_ref[:, 0:1]
    mask = cnt > 0.0
    inv = jnp.where(mask, 1.0 / jnp.where(mask, cnt, 1.0), 0.0)
    c = centers_ref[...]
    fs = jnp.concatenate([f0_ref[0], f1_ref[0]], axis=-1)
    nc_ref[...] = c - _ALPHA * jnp.where(mask, c - fs * inv, 0.0)
    f = features_ref[...]
    p = jnp.sum(c * c * cnt) - 2.0 * jnp.sum(c * fs) + jnp.sum(f * f)
    acc_ref[0] = jnp.where(i == 0, p, acc_ref[0] + p)

    @pl.when(i == pl.num_programs(0) - 1)
    def _done():
        loss_ref[...] = jnp.reshape(acc_ref[0] * (1.0 / (_BATCH * _DIM)), (1, 1))


_dense = pl.pallas_call(
    _dense_body,
    grid=(_G,),
    in_specs=[
        pl.BlockSpec((_CB, 128), lambda i: (i, 0)),
        pl.BlockSpec((1, _CB, _COLS), lambda i: (0, i, 0)),
        pl.BlockSpec((1, _CB, _COLS), lambda i: (1, i, 0)),
        pl.BlockSpec((_CB, _DIM), lambda i: (i, 0)),
        pl.BlockSpec((_FB, _DIM), lambda i: (i, 0)),
    ],
    out_specs=[
        pl.BlockSpec((_CB, _DIM), lambda i: (i, 0)),
        pl.BlockSpec((1, 1), lambda i: (0, 0)),
    ],
    out_shape=[
        jax.ShapeDtypeStruct((_NUM_CLASSES, _DIM), jnp.float32),
        jax.ShapeDtypeStruct((1, 1), jnp.float32),
    ],
    scratch_shapes=[pltpu.SMEM((1,), jnp.float32)],
)


@jax.jit
def _impl(features, labels, centers):
    labels_i32 = labels.astype(jnp.int32)
    zeros_b = jnp.zeros((_FS_WPT, 128), jnp.float32)
    ones_b = jnp.ones((_CHUNK, 128), jnp.float32)
    featsum, counts = _sc_segsum(features, labels_i32, zeros_b, ones_b)
    counts2 = counts.reshape(_NUM_CLASSES, 128)
    new_centers, loss = _dense(counts2, featsum, featsum, centers, features)
    return loss[0, 0], new_centers


def kernel(features, labels, centers):
    return _impl(features, labels, centers)


# speedup vs baseline: 4.2072x; 1.2305x over previous
import jax
import jax.numpy as jnp
from jax import lax
from jax.experimental import pallas as pl
from jax.experimental.pallas import tpu as pltpu
from jax.experimental.pallas import tpu_sc as plsc

_NUM_CLASSES = 8192
_DIM = 256
_ALPHA = 0.5
_BATCH = 16384

_NC = 2
_NS = 16
_COLS = _DIM // _NC
_HALF = _NUM_CLASSES // _NC
_ROWS_PER_TILE = _BATCH // _NS
_CHUNK = 64
_NCHUNK = _ROWS_PER_TILE // _CHUNK
_CNT_ROWS = _HALF + _NS
_FS_WPT = _NUM_CLASSES // _NS
_CNT_WPT = _HALF // _NS


def _sc_body(features, labels, zeros_b, ones_b,
             featsum_out, counts_out,
             lbl_all, idxf0, idxf1, idx20, idx21, fbuf0, fbuf1, ones_ref,
             featsum_sh, counts_sh,
             lsem0, lsem1, fsem0, fsem1, csem0, csem1):
    c = lax.axis_index("c")
    s = lax.axis_index("s")
    col0 = c * _COLS
    base = c * _HALF
    row0 = s * _ROWS_PER_TILE
    trash = _HALF + s

    idxf = (idxf0, idxf1)
    idx2 = (idx20, idx21)
    fbuf = (fbuf0, fbuf1)
    lsem = (lsem0, lsem1)
    fsem = (fsem0, fsem1)
    csem = (csem0, csem1)

    pltpu.sync_copy(ones_b, ones_ref)
    pltpu.sync_copy(labels.at[pl.ds(row0, _ROWS_PER_TILE)], lbl_all)
    pltpu.sync_copy(zeros_b, featsum_sh.at[pl.ds(s * _FS_WPT, _FS_WPT)])
    pltpu.sync_copy(zeros_b.at[pl.ds(0, _CNT_WPT)],
                    counts_sh.at[pl.ds(s * _CNT_WPT, _CNT_WPT)])

    @pl.when(s == _NS - 1)
    def _zero_trash():
        pltpu.sync_copy(zeros_b.at[pl.ds(0, _NS)],
                        counts_sh.at[pl.ds(_HALF, _NS)])

    def _compute_idx(j, b):
        for k in range(_CHUNK // 16):
            lbl = lbl_all[pl.ds(j * _CHUNK + k * 16, 16)]
            idxf[b][pl.ds(k * 16, 16)] = lbl
            rel = lbl - base
            inr = jnp.logical_and(rel >= 0, rel < _HALF)
            idx2[b][pl.ds(k * 16, 16)] = jnp.where(inr, rel, trash)

    plsc.subcore_barrier()

    _compute_idx(0, 0)
    ld = [None, None]
    fs = [None, None]
    cs = [None, None]
    ld[0] = pltpu.async_copy(
        features.at[pl.ds(row0, _CHUNK), pl.ds(col0, _COLS)], fbuf[0],
        lsem[0])
    for j in range(_NCHUNK):
        b = j & 1
        nb = 1 - b
        if j + 1 < _NCHUNK:
            if fs[nb] is not None:
                fs[nb].wait()
                cs[nb].wait()
            _compute_idx(j + 1, nb)
            ld[nb] = pltpu.async_copy(
                features.at[pl.ds(row0 + (j + 1) * _CHUNK, _CHUNK),
                            pl.ds(col0, _COLS)],
                fbuf[nb], lsem[nb])
        ld[b].wait()
        fs[b] = pltpu.async_copy(fbuf[b], featsum_sh.at[idxf[b]], fsem[b],
                                 add=True)
        cs[b] = pltpu.async_copy(ones_ref, counts_sh.at[idx2[b]], csem[b],
                                 add=True)
    fs[0].wait()
    cs[0].wait()
    fs[1].wait()
    cs[1].wait()

    plsc.subcore_barrier()

    pltpu.sync_copy(featsum_sh.at[pl.ds(s * _FS_WPT, _FS_WPT)],
                    featsum_out.at[c, pl.ds(s * _FS_WPT, _FS_WPT)])
    pltpu.sync_copy(counts_sh.at[pl.ds(s * _CNT_WPT, _CNT_WPT)],
                    counts_out.at[c, pl.ds(s * _CNT_WPT, _CNT_WPT)])


_sc_segsum = pl.kernel(
    _sc_body,
    out_type=(
        jax.ShapeDtypeStruct((_NC, _NUM_CLASSES, _COLS), jnp.float32),
        jax.ShapeDtypeStruct((_NC, _HALF, 128), jnp.float32),
    ),
    mesh=plsc.VectorSubcoreMesh(core_axis_name="c", subcore_axis_name="s"),
    scratch_types=[
        pltpu.VMEM((_ROWS_PER_TILE,), jnp.int32),
        pltpu.VMEM((_CHUNK,), jnp.int32),
        pltpu.VMEM((_CHUNK,), jnp.int32),
        pltpu.VMEM((_CHUNK,), jnp.int32),
        pltpu.VMEM((_CHUNK,), jnp.int32),
        pltpu.VMEM((_CHUNK, _COLS), jnp.float32),
        pltpu.VMEM((_CHUNK, _COLS), jnp.float32),
        pltpu.VMEM((_CHUNK, 128), jnp.float32),
        pltpu.VMEM_SHARED((_NUM_CLASSES, _COLS), jnp.float32),
        pltpu.VMEM_SHARED((_CNT_ROWS, 128), jnp.float32),
        pltpu.SemaphoreType.DMA,
        pltpu.SemaphoreType.DMA,
        pltpu.SemaphoreType.DMA,
        pltpu.SemaphoreType.DMA,
        pltpu.SemaphoreType.DMA,
        pltpu.SemaphoreType.DMA,
    ],
)


_G = 16
_CB = _NUM_CLASSES // _G
_FB = _BATCH // _G


def _dense_body(counts_ref, f0_ref, f1_ref, centers_ref, features_ref,
                nc_ref, loss_ref, acc_ref):
    i = pl.program_id(0)
    cnt = counts_ref[:, 0:1]
    mask = cnt > 0.0
    inv = jnp.where(mask, 1.0 / jnp.where(mask, cnt, 1.0), 0.0)
    c = centers_ref[...]
    fs = jnp.concatenate([f0_ref[0], f1_ref[0]], axis=-1)
    nc_ref[...] = c - _ALPHA * jnp.where(mask, c - fs * inv, 0.0)
    f = features_ref[...]
    p = jnp.sum(c * c * cnt) - 2.0 * jnp.sum(c * fs) + jnp.sum(f * f)
    acc_ref[0] = jnp.where(i == 0, p, acc_ref[0] + p)

    @pl.when(i == pl.num_programs(0) - 1)
    def _done():
        loss_ref[...] = jnp.reshape(acc_ref[0] * (1.0 / (_BATCH * _DIM)), (1, 1))


_dense = pl.pallas_call(
    _dense_body,
    grid=(_G,),
    in_specs=[
        pl.BlockSpec((_CB, 128), lambda i: (i, 0)),
        pl.BlockSpec((1, _CB, _COLS), lambda i: (0, i, 0)),
        pl.BlockSpec((1, _CB, _COLS), lambda i: (1, i, 0)),
        pl.BlockSpec((_CB, _DIM), lambda i: (i, 0)),
        pl.BlockSpec((_FB, _DIM), lambda i: (i, 0)),
    ],
    out_specs=[
        pl.BlockSpec((_CB, _DIM), lambda i: (i, 0)),
        pl.BlockSpec((1, 1), lambda i: (0, 0)),
    ],
    out_shape=[
        jax.ShapeDtypeStruct((_NUM_CLASSES, _DIM), jnp.float32),
        jax.ShapeDtypeStruct((1, 1), jnp.float32),
    ],
    scratch_shapes=[pltpu.SMEM((1,), jnp.float32)],
)


@jax.jit
def _impl(features, labels, centers):
    labels_i32 = labels.astype(jnp.int32)
    zeros_b = jnp.zeros((_FS_WPT, 128), jnp.float32)
    ones_b = jnp.ones((_CHUNK, 128), jnp.float32)
    featsum, counts = _sc_segsum(features, labels_i32, zeros_b, ones_b)
    counts2 = counts.reshape(_NUM_CLASSES, 128)
    new_centers, loss = _dense(counts2, featsum, featsum, centers, features)
    return loss[0, 0], new_centers


def kernel(features, labels, centers):
    return _impl(features, labels, centers)


# speedup vs baseline: 4.8724x; 1.1581x over previous
import jax
import jax.numpy as jnp
from jax import lax
from jax.experimental import pallas as pl
from jax.experimental.pallas import tpu as pltpu
from jax.experimental.pallas import tpu_sc as plsc

_NUM_CLASSES = 8192
_DIM = 256
_ALPHA = 0.5
_BATCH = 16384

_NC = 2
_NS = 16
_COLS = _DIM // _NC
_HALF = _NUM_CLASSES // _NC
_ROWS_PER_TILE = _BATCH // _NS
_CHUNK = 64
_NCHUNK = _ROWS_PER_TILE // _CHUNK
_CNT_ROWS = _HALF + _NS
_FS_WPT = _NUM_CLASSES // _NS
_CNT_WPT = _HALF // _NS


def _sc_body(features, labels,
             featsum_out, counts_out,
             lbl_all, idxf0, idxf1, idx20, idx21, fbuf0, fbuf1, ones_ref,
             featsum_sh, counts_sh,
             lsem0, lsem1, fsem0, fsem1, csem0, csem1):
    c = lax.axis_index("c")
    s = lax.axis_index("s")
    col0 = c * _COLS
    base = c * _HALF
    row0 = s * _ROWS_PER_TILE
    trash = _HALF + s

    idxf = (idxf0, idxf1)
    idx2 = (idx20, idx21)
    fbuf = (fbuf0, fbuf1)
    lsem = (lsem0, lsem1)
    fsem = (fsem0, fsem1)
    csem = (csem0, csem1)

    zeros16 = jnp.zeros((16,), jnp.float32)
    ones16 = jnp.ones((16,), jnp.float32)

    def _fill(i, carry):
        for k in range(128 // 16):
            fbuf0[i, pl.ds(k * 16, 16)] = zeros16
            ones_ref[i, pl.ds(k * 16, 16)] = ones16
        return carry

    lax.fori_loop(0, _CHUNK, _fill, 0)
    pltpu.sync_copy(labels.at[pl.ds(row0, _ROWS_PER_TILE)], lbl_all)
    for k in range(_FS_WPT // _CHUNK):
        pltpu.sync_copy(fbuf0,
                        featsum_sh.at[pl.ds(s * _FS_WPT + k * _CHUNK, _CHUNK)])
    for k in range(_CNT_WPT // _CHUNK):
        pltpu.sync_copy(fbuf0,
                        counts_sh.at[pl.ds(s * _CNT_WPT + k * _CHUNK, _CHUNK)])

    @pl.when(s == _NS - 1)
    def _zero_trash():
        pltpu.sync_copy(fbuf0.at[pl.ds(0, _NS)],
                        counts_sh.at[pl.ds(_HALF, _NS)])

    def _compute_idx(j, b):
        for k in range(_CHUNK // 16):
            lbl = lbl_all[pl.ds(j * _CHUNK + k * 16, 16)]
            idxf[b][pl.ds(k * 16, 16)] = lbl
            rel = lbl - base
            inr = jnp.logical_and(rel >= 0, rel < _HALF)
            idx2[b][pl.ds(k * 16, 16)] = jnp.where(inr, rel, trash)

    plsc.subcore_barrier()

    _compute_idx(0, 0)
    ld = [None, None]
    fs = [None, None]
    cs = [None, None]
    ld[0] = pltpu.async_copy(
        features.at[pl.ds(row0, _CHUNK), pl.ds(col0, _COLS)], fbuf[0],
        lsem[0])
    for j in range(_NCHUNK):
        b = j & 1
        nb = 1 - b
        if j + 1 < _NCHUNK:
            if fs[nb] is not None:
                fs[nb].wait()
                cs[nb].wait()
            _compute_idx(j + 1, nb)
            ld[nb] = pltpu.async_copy(
                features.at[pl.ds(row0 + (j + 1) * _CHUNK, _CHUNK),
                            pl.ds(col0, _COLS)],
                fbuf[nb], lsem[nb])
        ld[b].wait()
        fs[b] = pltpu.async_copy(fbuf[b], featsum_sh.at[idxf[b]], fsem[b],
                                 add=True)
        cs[b] = pltpu.async_copy(ones_ref, counts_sh.at[idx2[b]], csem[b],
                                 add=True)
    fs[0].wait()
    cs[0].wait()
    fs[1].wait()
    cs[1].wait()

    plsc.subcore_barrier()

    pltpu.sync_copy(featsum_sh.at[pl.ds(s * _FS_WPT, _FS_WPT)],
                    featsum_out.at[c, pl.ds(s * _FS_WPT, _FS_WPT)])
    pltpu.sync_copy(counts_sh.at[pl.ds(s * _CNT_WPT, _CNT_WPT)],
                    counts_out.at[c, pl.ds(s * _CNT_WPT, _CNT_WPT)])


_sc_segsum = pl.kernel(
    _sc_body,
    out_type=(
        jax.ShapeDtypeStruct((_NC, _NUM_CLASSES, _COLS), jnp.float32),
        jax.ShapeDtypeStruct((_NC, _HALF, 128), jnp.float32),
    ),
    mesh=plsc.VectorSubcoreMesh(core_axis_name="c", subcore_axis_name="s"),
    scratch_types=[
        pltpu.VMEM((_ROWS_PER_TILE,), jnp.int32),
        pltpu.VMEM((_CHUNK,), jnp.int32),
        pltpu.VMEM((_CHUNK,), jnp.int32),
        pltpu.VMEM((_CHUNK,), jnp.int32),
        pltpu.VMEM((_CHUNK,), jnp.int32),
        pltpu.VMEM((_CHUNK, _COLS), jnp.float32),
        pltpu.VMEM((_CHUNK, _COLS), jnp.float32),
        pltpu.VMEM((_CHUNK, 128), jnp.float32),
        pltpu.VMEM_SHARED((_NUM_CLASSES, _COLS), jnp.float32),
        pltpu.VMEM_SHARED((_CNT_ROWS, 128), jnp.float32),
        pltpu.SemaphoreType.DMA,
        pltpu.SemaphoreType.DMA,
        pltpu.SemaphoreType.DMA,
        pltpu.SemaphoreType.DMA,
        pltpu.SemaphoreType.DMA,
        pltpu.SemaphoreType.DMA,
    ],
)


_G = 16
_CB = _NUM_CLASSES // _G
_FB = _BATCH // _G


def _f2_body(features_ref, out_ref, acc_ref):
    i = pl.program_id(0)
    f = features_ref[...]
    p = jnp.sum(f * f)
    acc_ref[0] = jnp.where(i == 0, p, acc_ref[0] + p)

    @pl.when(i == pl.num_programs(0) - 1)
    def _done():
        out_ref[...] = jnp.reshape(acc_ref[0], (1, 1))


_f2 = pl.pallas_call(
    _f2_body,
    grid=(_G,),
    in_specs=[pl.BlockSpec((_FB, _DIM), lambda i: (i, 0))],
    out_specs=[pl.BlockSpec((1, 1), lambda i: (0, 0))],
    out_shape=[jax.ShapeDtypeStruct((1, 1), jnp.float32)],
    scratch_shapes=[pltpu.SMEM((1,), jnp.float32)],
)


def _dense_body(counts_ref, f0_ref, f1_ref, centers_ref, s3_ref,
                nc_ref, loss_ref, acc_ref):
    i = pl.program_id(0)
    cnt = counts_ref[:, 0:1]
    mask = cnt > 0.0
    inv = jnp.where(mask, 1.0 / jnp.where(mask, cnt, 1.0), 0.0)
    c = centers_ref[...]
    fs = jnp.concatenate([f0_ref[0], f1_ref[0]], axis=-1)
    nc_ref[...] = c - _ALPHA * jnp.where(mask, c - fs * inv, 0.0)
    p = jnp.sum(c * c * cnt) - 2.0 * jnp.sum(c * fs)
    acc_ref[0] = jnp.where(i == 0, p, acc_ref[0] + p)

    @pl.when(i == pl.num_programs(0) - 1)
    def _done():
        loss_ref[...] = jnp.reshape(
            (acc_ref[0] + s3_ref[0, 0]) * (1.0 / (_BATCH * _DIM)), (1, 1))


_dense = pl.pallas_call(
    _dense_body,
    grid=(_G,),
    in_specs=[
        pl.BlockSpec((_CB, 128), lambda i: (i, 0)),
        pl.BlockSpec((1, _CB, _COLS), lambda i: (0, i, 0)),
        pl.BlockSpec((1, _CB, _COLS), lambda i: (1, i, 0)),
        pl.BlockSpec((_CB, _DIM), lambda i: (i, 0)),
        pl.BlockSpec((1, 1), lambda i: (0, 0)),
    ],
    out_specs=[
        pl.BlockSpec((_CB, _DIM), lambda i: (i, 0)),
        pl.BlockSpec((1, 1), lambda i: (0, 0)),
    ],
    out_shape=[
        jax.ShapeDtypeStruct((_NUM_CLASSES, _DIM), jnp.float32),
        jax.ShapeDtypeStruct((1, 1), jnp.float32),
    ],
    scratch_shapes=[pltpu.SMEM((1,), jnp.float32)],
)


@jax.jit
def _impl(features, labels, centers):
    labels_i32 = labels.astype(jnp.int32)
    (s3,) = _f2(features)
    featsum, counts = _sc_segsum(features, labels_i32)
    counts2 = counts.reshape(_NUM_CLASSES, 128)
    new_centers, loss = _dense(counts2, featsum, featsum, centers, s3)
    return loss[0, 0], new_centers


def kernel(features, labels, centers):
    return _impl(features, labels, centers)


# speedup vs baseline: 4.9111x; 1.0079x over previous
import jax
import jax.numpy as jnp
from jax import lax
from jax.experimental import pallas as pl
from jax.experimental.pallas import tpu as pltpu
from jax.experimental.pallas import tpu_sc as plsc

_NUM_CLASSES = 8192
_DIM = 256
_ALPHA = 0.5
_BATCH = 16384

_NC = 2
_NS = 16
_COLS = _DIM // _NC
_HALF = _NUM_CLASSES // _NC
_ROWS_PER_TILE = _BATCH // _NS
_CHUNK = 64
_NCHUNK = _ROWS_PER_TILE // _CHUNK
_CNT_ROWS = _HALF + _NS
_FS_WPT = _NUM_CLASSES // _NS
_CNT_WPT = _HALF // _NS


def _sc_body(features, labels,
             featsum_out, counts_out,
             lbl_all, idxf0, idxf1, idx20, idx21, fbuf0, fbuf1, ones_ref,
             featsum_sh, counts_sh,
             lsem0, lsem1, fsem0, fsem1, csem0, csem1):
    c = lax.axis_index("c")
    s = lax.axis_index("s")
    col0 = c * _COLS
    base = c * _HALF
    row0 = s * _ROWS_PER_TILE
    trash = _HALF + s

    idxf = (idxf0, idxf1)
    idx2 = (idx20, idx21)
    fbuf = (fbuf0, fbuf1)
    lsem = (lsem0, lsem1)
    fsem = (fsem0, fsem1)
    csem = (csem0, csem1)

    zeros16 = jnp.zeros((16,), jnp.float32)
    ones16 = jnp.ones((16,), jnp.float32)

    def _fill(i, carry):
        for k in range(128 // 16):
            fbuf0[i, pl.ds(k * 16, 16)] = zeros16
            ones_ref[i, pl.ds(k * 16, 16)] = ones16
        return carry

    lax.fori_loop(0, _CHUNK, _fill, 0)
    zw = []
    zw.append(pltpu.async_copy(labels.at[pl.ds(row0, _ROWS_PER_TILE)],
                               lbl_all, lsem0))
    for k in range(_FS_WPT // _CHUNK):
        zw.append(pltpu.async_copy(
            fbuf0, featsum_sh.at[pl.ds(s * _FS_WPT + k * _CHUNK, _CHUNK)],
            fsem0))
    for k in range(_CNT_WPT // _CHUNK):
        zw.append(pltpu.async_copy(
            fbuf0, counts_sh.at[pl.ds(s * _CNT_WPT + k * _CHUNK, _CHUNK)],
            csem0))

    @pl.when(s == _NS - 1)
    def _zero_trash():
        pltpu.sync_copy(fbuf0.at[pl.ds(0, _NS)],
                        counts_sh.at[pl.ds(_HALF, _NS)])

    for w in zw:
        w.wait()

    def _compute_idx(j, b):
        for k in range(_CHUNK // 16):
            lbl = lbl_all[pl.ds(j * _CHUNK + k * 16, 16)]
            idxf[b][pl.ds(k * 16, 16)] = lbl
            rel = lbl - base
            inr = jnp.logical_and(rel >= 0, rel < _HALF)
            idx2[b][pl.ds(k * 16, 16)] = jnp.where(inr, rel, trash)

    plsc.subcore_barrier()

    _compute_idx(0, 0)
    ld = [None, None]
    fs = [None, None]
    cs = [None, None]
    ld[0] = pltpu.async_copy(
        features.at[pl.ds(row0, _CHUNK), pl.ds(col0, _COLS)], fbuf[0],
        lsem[0])
    for j in range(_NCHUNK):
        b = j & 1
        nb = 1 - b
        if j + 1 < _NCHUNK:
            if fs[nb] is not None:
                fs[nb].wait()
                cs[nb].wait()
            _compute_idx(j + 1, nb)
            ld[nb] = pltpu.async_copy(
                features.at[pl.ds(row0 + (j + 1) * _CHUNK, _CHUNK),
                            pl.ds(col0, _COLS)],
                fbuf[nb], lsem[nb])
        ld[b].wait()
        fs[b] = pltpu.async_copy(fbuf[b], featsum_sh.at[idxf[b]], fsem[b],
                                 add=True)
        cs[b] = pltpu.async_copy(ones_ref, counts_sh.at[idx2[b]], csem[b],
                                 add=True)
    fs[0].wait()
    cs[0].wait()
    fs[1].wait()
    cs[1].wait()

    plsc.subcore_barrier()

    w1 = pltpu.async_copy(featsum_sh.at[pl.ds(s * _FS_WPT, _FS_WPT)],
                          featsum_out.at[c, pl.ds(s * _FS_WPT, _FS_WPT)],
                          lsem0)
    w2 = pltpu.async_copy(counts_sh.at[pl.ds(s * _CNT_WPT, _CNT_WPT)],
                          counts_out.at[c, pl.ds(s * _CNT_WPT, _CNT_WPT)],
                          lsem1)
    w1.wait()
    w2.wait()


_sc_segsum = pl.kernel(
    _sc_body,
    out_type=(
        jax.ShapeDtypeStruct((_NC, _NUM_CLASSES, _COLS), jnp.float32),
        jax.ShapeDtypeStruct((_NC, _HALF, 128), jnp.float32),
    ),
    mesh=plsc.VectorSubcoreMesh(core_axis_name="c", subcore_axis_name="s"),
    scratch_types=[
        pltpu.VMEM((_ROWS_PER_TILE,), jnp.int32),
        pltpu.VMEM((_CHUNK,), jnp.int32),
        pltpu.VMEM((_CHUNK,), jnp.int32),
        pltpu.VMEM((_CHUNK,), jnp.int32),
        pltpu.VMEM((_CHUNK,), jnp.int32),
        pltpu.VMEM((_CHUNK, _COLS), jnp.float32),
        pltpu.VMEM((_CHUNK, _COLS), jnp.float32),
        pltpu.VMEM((_CHUNK, 128), jnp.float32),
        pltpu.VMEM_SHARED((_NUM_CLASSES, _COLS), jnp.float32),
        pltpu.VMEM_SHARED((_CNT_ROWS, 128), jnp.float32),
        pltpu.SemaphoreType.DMA,
        pltpu.SemaphoreType.DMA,
        pltpu.SemaphoreType.DMA,
        pltpu.SemaphoreType.DMA,
        pltpu.SemaphoreType.DMA,
        pltpu.SemaphoreType.DMA,
    ],
)


_G = 16
_CB = _NUM_CLASSES // _G
_FB = _BATCH // _G


def _f2_body(features_ref, out_ref, acc_ref):
    i = pl.program_id(0)
    f = features_ref[...]
    p = jnp.sum(f * f)
    acc_ref[0] = jnp.where(i == 0, p, acc_ref[0] + p)

    @pl.when(i == pl.num_programs(0) - 1)
    def _done():
        out_ref[...] = jnp.reshape(acc_ref[0], (1, 1))


_f2 = pl.pallas_call(
    _f2_body,
    grid=(_G,),
    in_specs=[pl.BlockSpec((_FB, _DIM), lambda i: (i, 0))],
    out_specs=[pl.BlockSpec((1, 1), lambda i: (0, 0))],
    out_shape=[jax.ShapeDtypeStruct((1, 1), jnp.float32)],
    scratch_shapes=[pltpu.SMEM((1,), jnp.float32)],
)


def _dense_body(counts_ref, f0_ref, f1_ref, centers_ref, s3_ref,
                nc_ref, loss_ref, acc_ref):
    i = pl.program_id(0)
    cnt = counts_ref[:, 0:1]
    mask = cnt > 0.0
    inv = jnp.where(mask, 1.0 / jnp.where(mask, cnt, 1.0), 0.0)
    c = centers_ref[...]
    fs = jnp.concatenate([f0_ref[0], f1_ref[0]], axis=-1)
    nc_ref[...] = c - _ALPHA * jnp.where(mask, c - fs * inv, 0.0)
    p = jnp.sum(c * c * cnt) - 2.0 * jnp.sum(c * fs)
    acc_ref[0] = jnp.where(i == 0, p, acc_ref[0] + p)

    @pl.when(i == pl.num_programs(0) - 1)
    def _done():
        loss_ref[...] = jnp.reshape(
            (acc_ref[0] + s3_ref[0, 0]) * (1.0 / (_BATCH * _DIM)), (1, 1))


_dense = pl.pallas_call(
    _dense_body,
    grid=(_G,),
    in_specs=[
        pl.BlockSpec((_CB, 128), lambda i: (i, 0)),
        pl.BlockSpec((1, _CB, _COLS), lambda i: (0, i, 0)),
        pl.BlockSpec((1, _CB, _COLS), lambda i: (1, i, 0)),
        pl.BlockSpec((_CB, _DIM), lambda i: (i, 0)),
        pl.BlockSpec((1, 1), lambda i: (0, 0)),
    ],
    out_specs=[
        pl.BlockSpec((_CB, _DIM), lambda i: (i, 0)),
        pl.BlockSpec((1, 1), lambda i: (0, 0)),
    ],
    out_shape=[
        jax.ShapeDtypeStruct((_NUM_CLASSES, _DIM), jnp.float32),
        jax.ShapeDtypeStruct((1, 1), jnp.float32),
    ],
    scratch_shapes=[pltpu.SMEM((1,), jnp.float32)],
)


@jax.jit
def _impl(features, labels, centers):
    labels_i32 = labels.astype(jnp.int32)
    (s3,) = _f2(features)
    featsum, counts = _sc_segsum(features, labels_i32)
    counts2 = counts.reshape(_NUM_CLASSES, 128)
    new_centers, loss = _dense(counts2, featsum, featsum, centers, s3)
    return loss[0, 0], new_centers


def kernel(features, labels, centers):
    return _impl(features, labels, centers)
